# baseline (speedup 1.0000x reference)
import math

import jax
import jax.numpy as jnp
from jax.experimental import pallas as pl
from jax.experimental.pallas import tpu as pltpu

_SCALE = 400.0 / math.log(10.0)
_DEFAULT_PRED = 7.6699353278706015
_NEG = -1e30

_TOTAL = 32768
_D = 256
_H = 32
_B = 16
_BLK = 2048
_GRID = _TOTAL // _BLK


def _fused_kernel(x_ref, w1t_ref, wrz_ref, meta_ref, out_ref, acc_ref):
    i = pl.program_id(0)

    @pl.when(i == 0)
    def _init():
        acc_ref[0:1, :] = jnp.full((1, _B), _NEG, jnp.float32)
        acc_ref[1:2, :] = jnp.zeros((1, _B), jnp.float32)
        acc_ref[2:3, :] = jnp.zeros((1, _B), jnp.float32)

    xb = x_ref[:]
    b1 = meta_ref[0:1, :]
    brz = meta_ref[1:2, 0:2]
    starts = meta_ref[2:3, 0:_B]
    ends = meta_ref[3:4, 0:_B]

    hb = jnp.maximum(
        jnp.dot(xb, w1t_ref[:], preferred_element_type=jnp.float32) + b1, 0.0
    )
    rz = jnp.dot(hb, wrz_ref[:], preferred_element_type=jnp.float32) + brz
    r = rz[:, 0:1]
    z = rz[:, 1:2]

    idx = (
        jax.lax.broadcasted_iota(jnp.int32, (_BLK, 1), 0) + i * _BLK
    ).astype(jnp.float32)
    mask = (idx >= starts) & (idx < ends)
    zm = jnp.where(mask, z, _NEG)

    old_max = acc_ref[0:1, :]
    blk_max = jnp.max(zm, axis=0, keepdims=True)
    new_max = jnp.maximum(old_max, blk_max)
    scale = jnp.exp(old_max - new_max)

    e = jnp.exp(zm - new_max) * mask.astype(jnp.float32)
    s = jnp.sum(e, axis=0, keepdims=True)
    sr = jnp.sum(e * r, axis=0, keepdims=True)

    acc_ref[0:1, :] = new_max
    acc_ref[1:2, :] = acc_ref[1:2, :] * scale + s
    acc_ref[2:3, :] = acc_ref[2:3, :] * scale + sr

    @pl.when(i == _GRID - 1)
    def _finish():
        denom = acc_ref[1:2, :]
        preds = acc_ref[2:3, :] / jnp.where(denom == 0.0, 1.0, denom)
        empty = starts == ends
        preds = jnp.where(empty, _DEFAULT_PRED, preds)
        out_ref[:] = _SCALE * preds


def kernel(x, xlens, W1, b1, Wr, br, Wz, bz):
    w1t = W1.T
    wrz = jnp.concatenate([Wr, Wz], axis=0).T

    clens = jnp.concatenate(
        [jnp.zeros((1,), dtype=xlens.dtype), jnp.cumsum(xlens)]
    )
    starts = clens[:-1].astype(jnp.float32)
    ends = clens[1:].astype(jnp.float32)

    meta = jnp.zeros((8, _H), jnp.float32)
    meta = meta.at[0, :].set(b1)
    meta = meta.at[1, 0].set(br[0])
    meta = meta.at[1, 1].set(bz[0])
    meta = meta.at[2, :_B].set(starts)
    meta = meta.at[3, :_B].set(ends)

    out = pl.pallas_call(
        _fused_kernel,
        grid=(_GRID,),
        in_specs=[
            pl.BlockSpec((_BLK, _D), lambda i: (i, 0)),
            pl.BlockSpec((_D, _H), lambda i: (0, 0)),
            pl.BlockSpec((_H, 2), lambda i: (0, 0)),
            pl.BlockSpec((8, _H), lambda i: (0, 0)),
        ],
        out_specs=pl.BlockSpec((1, _B), lambda i: (0, 0)),
        out_shape=jax.ShapeDtypeStruct((1, _B), jnp.float32),
        scratch_shapes=[pltpu.VMEM((8, _B), jnp.float32)],
    )(x, w1t, wrz, meta)
    return out.reshape(_B)
